# baseline (speedup 1.0000x reference)
import functools

import jax
import jax.numpy as jnp
from jax import lax
from jax.experimental import pallas as pl
from jax.experimental.pallas import tpu as pltpu
from jax.experimental.pallas import tpu_sc as plsc

N = 10000
E = 320000
D = 128
NC = 2
NS = 16
NW = NC * NS
EPW = E // NW
K = 80
NBLK = EPW // K
NBUF = 3
W = 25
WK = W * K
NCHUNK = NBLK // W
CP = 624
REM = N - CP * NS

_mesh = plsc.VectorSubcoreMesh(core_axis_name="c", subcore_axis_name="s")
_sc_params = pltpu.CompilerParams(needs_layout_passes=False)


def _hist_body(col_hbm, hist_out, col_v, hist_v):
    c = lax.axis_index("c")
    s = lax.axis_index("s")
    wid = c * NS + s
    pltpu.sync_copy(col_hbm.at[wid], col_v)

    def zero(i, carry):
        hist_v[pl.ds(pl.multiple_of(i * 16, 16), 16)] = jnp.zeros(
            (16,), jnp.float32)
        return carry
    lax.fori_loop(0, N // 16, zero, 0)

    ones = jnp.ones((16,), jnp.float32)

    def body(i, carry):
        cv = col_v[pl.ds(pl.multiple_of(i * 16, 16), 16)]
        plsc.addupdate_scatter(hist_v, [cv], ones)
        return carry
    lax.fori_loop(0, EPW // 16, body, 0)
    pltpu.sync_copy(hist_v, hist_out.at[wid])


_hist_kernel = functools.partial(
    pl.kernel,
    out_type=jax.ShapeDtypeStruct((NW, N), jnp.float32),
    mesh=_mesh,
    compiler_params=_sc_params,
    scratch_types=[
        pltpu.VMEM((EPW,), jnp.int32),
        pltpu.VMEM((N,), jnp.float32),
    ],
)(_hist_body)


def _enc_mm_body(x_ref, w_ref, b_ref, h_ref):
    h = lax.dot_general(x_ref[...], w_ref[...],
                        (((1,), (1,)), ((), ())),
                        precision=lax.Precision.HIGHEST,
                        preferred_element_type=jnp.float32)
    h_ref[...] = jnp.maximum(h + b_ref[...], 0.0)


def _enc_mm(x, w, b):
    return pl.pallas_call(
        _enc_mm_body,
        out_shape=jax.ShapeDtypeStruct((N, D), jnp.float32),
    )(x, w, b)


def _scale_body(h_ref, hist_ref, hs_ref, dis_ref):
    deg = jnp.sum(hist_ref[...], axis=0) + 1.0
    dis = deg ** -0.5
    hs_ref[...] = h_ref[...] * dis[:, None]
    dis_ref[...] = dis[:, None]


def _scale(h, hist):
    return pl.pallas_call(
        _scale_body,
        out_shape=(jax.ShapeDtypeStruct((N, D), jnp.float32),
                   jax.ShapeDtypeStruct((N, 1), jnp.float32)),
    )(h, hist)


def _scat_body(hs_hbm, idx_hbm, zero_hbm, out_hbm,
               idx_v, rows_v, acc, gsems, ssems, isem):
    c = lax.axis_index("c")
    s = lax.axis_index("s")
    wid = c * NS + s
    base = s * CP
    pltpu.sync_copy(zero_hbm.at[pl.ds(base, CP)], acc.at[pl.ds(base, CP)])

    @pl.when(s == 0)
    def _():
        pltpu.sync_copy(zero_hbm.at[pl.ds(CP * NS, REM)],
                        acc.at[pl.ds(CP * NS, REM)])

    def _chunk_fire(ch, sync=False):
        r = (wid * NCHUNK + ch) * 2
        p = (ch % 2) * 2
        if sync:
            pltpu.sync_copy(idx_hbm.at[r], idx_v.at[p])
            pltpu.sync_copy(idx_hbm.at[r + 1], idx_v.at[p + 1])
        else:
            pltpu.async_copy(idx_hbm.at[r], idx_v.at[p], isem)
            pltpu.async_copy(idx_hbm.at[r + 1], idx_v.at[p + 1], isem)

    def _chunk_wait(ch):
        r = (wid * NCHUNK + ch) * 2
        p = (ch % 2) * 2
        pltpu.make_async_copy(idx_hbm.at[r], idx_v.at[p], isem).wait()
        pltpu.make_async_copy(idx_hbm.at[r + 1], idx_v.at[p + 1], isem).wait()

    _chunk_fire(0, sync=True)
    _chunk_fire(1)
    plsc.subcore_barrier()

    def _rowidx(j):
        return idx_v.at[((j // W) % 2) * 2, j % W]

    def _colidx(j):
        return idx_v.at[((j // W) % 2) * 2 + 1, j % W]

    def _gfire(j, b):
        pltpu.async_copy(hs_hbm.at[_rowidx(j)], rows_v.at[b], gsems.at[b])

    def _gwait(j, b):
        pltpu.make_async_copy(hs_hbm.at[_rowidx(j)], rows_v.at[b],
                              gsems.at[b]).wait()

    def _sfire(j, b):
        pltpu.async_copy(rows_v.at[b], acc.at[_colidx(j)], ssems.at[b],
                         add=True)

    def _swait(j, b):
        pltpu.make_async_copy(rows_v.at[b], acc.at[_colidx(j)],
                              ssems.at[b]).wait()

    def _visit(j, b):
        _gwait(j, b)
        _sfire(j, b)
        bp = (b + NBUF - 1) % NBUF

        @pl.when(j >= 1)
        def _():
            _swait(j - 1, bp)

        jf = j + NBUF - 1

        @pl.when(jnp.logical_and(jf % W == 0, jf < NBLK))
        def _():
            _chunk_wait(jf // W)

        @pl.when(jf < NBLK)
        def _():
            _gfire(jf, bp)

        @pl.when(jnp.logical_and(
            j % W == 1,
            jnp.logical_and(j // W >= 1, j // W <= NCHUNK - 2)))
        def _():
            _chunk_fire(j // W + 1)

    for b in range(NBUF - 1):
        _gfire(b, b)

    def group(g, carry):
        for b in range(NBUF):
            _visit(g * NBUF + b, b)
        return carry
    nfull = (NBLK // NBUF) * NBUF
    lax.fori_loop(0, NBLK // NBUF, group, 0)
    for jt in range(nfull, NBLK):
        _visit(jnp.int32(jt), jt % NBUF)
    _swait(jnp.int32(NBLK - 1), (NBLK - 1) % NBUF)

    plsc.subcore_barrier()
    pltpu.sync_copy(acc.at[pl.ds(base, CP)],
                    out_hbm.at[c, pl.ds(base, CP)])

    @pl.when(s == 0)
    def _():
        pltpu.sync_copy(acc.at[pl.ds(CP * NS, REM)],
                        out_hbm.at[c, pl.ds(CP * NS, REM)])


_scat_kernel = functools.partial(
    pl.kernel,
    out_type=jax.ShapeDtypeStruct((NC, N, D), jnp.float32),
    mesh=_mesh,
    compiler_params=_sc_params,
    scratch_types=[
        pltpu.VMEM((4, W, K), jnp.int32),
        pltpu.VMEM((NBUF, K, D), jnp.float32),
        pltpu.VMEM_SHARED((N, D), jnp.float32),
        pltpu.SemaphoreType.DMA((NBUF,)),
        pltpu.SemaphoreType.DMA((NBUF,)),
        pltpu.SemaphoreType.DMA,
    ],
)(_scat_body)


def _dec_body(p_ref, hs_ref, dis_ref, gb_ref, w_ref, b_ref, y_ref):
    m = (p_ref[0] + p_ref[1] + hs_ref[...]) * dis_ref[...] + gb_ref[...]
    y = lax.dot_general(m, w_ref[...], (((1,), (1,)), ((), ())),
                        precision=lax.Precision.HIGHEST,
                        preferred_element_type=jnp.float32)
    y_ref[...] = jax.nn.sigmoid(y + b_ref[...])


def _dec(p, hs, dis, gb, w, b):
    return pl.pallas_call(
        _dec_body,
        out_shape=jax.ShapeDtypeStruct((N, D), jnp.float32),
    )(p, hs, dis, gb, w, b)


def kernel(x, edge_index, W_enc, b_enc, W_dec, b_dec, gcn_bias):
    row = edge_index[0]
    col = edge_index[1]
    col_flat = col.reshape(NW, EPW)
    idx3 = jnp.concatenate(
        [row.reshape(NW, NCHUNK, 1, W, K), col.reshape(NW, NCHUNK, 1, W, K)],
        axis=2).reshape(NW * NCHUNK * 2, W, K)

    hist = _hist_kernel(col_flat)
    h = _enc_mm(x, W_enc, b_enc.reshape(1, D))
    hs, dis = _scale(h, hist)
    partials = _scat_kernel(hs, idx3, jnp.zeros((N, D), jnp.float32))
    return _dec(partials, hs, dis, gcn_bias.reshape(1, D), W_dec,
                b_dec.reshape(1, D))


# speedup vs baseline: 1.0499x; 1.0499x over previous
import functools

import jax
import jax.numpy as jnp
from jax import lax
from jax.experimental import pallas as pl
from jax.experimental.pallas import tpu as pltpu
from jax.experimental.pallas import tpu_sc as plsc

N = 10000
E = 320000
D = 128
NC = 2
NS = 16
NW = NC * NS
EPW = E // NW
K = 80
NBLK = EPW // K
NBUF = 3
W = 25
WK = W * K
NCHUNK = NBLK // W
CP = 624
REM = N - CP * NS

_mesh = plsc.VectorSubcoreMesh(core_axis_name="c", subcore_axis_name="s")
_sc_params = pltpu.CompilerParams(needs_layout_passes=False)


def _hist_body(col_hbm, hist_out, col_v, hist_v):
    c = lax.axis_index("c")
    s = lax.axis_index("s")
    wid = c * NS + s
    pltpu.sync_copy(col_hbm.at[wid], col_v)

    def zero(i, carry):
        hist_v[pl.ds(pl.multiple_of(i * 16, 16), 16)] = jnp.zeros(
            (16,), jnp.float32)
        return carry
    lax.fori_loop(0, N // 16, zero, 0)

    ones = jnp.ones((16,), jnp.float32)

    def body(i, carry):
        cv = col_v[pl.ds(pl.multiple_of(i * 16, 16), 16)]
        plsc.addupdate_scatter(hist_v, [cv], ones)
        return carry
    lax.fori_loop(0, EPW // 16, body, 0)
    pltpu.sync_copy(hist_v, hist_out.at[wid])


_hist_kernel = functools.partial(
    pl.kernel,
    out_type=jax.ShapeDtypeStruct((NW, N), jnp.float32),
    mesh=_mesh,
    compiler_params=_sc_params,
    scratch_types=[
        pltpu.VMEM((EPW,), jnp.int32),
        pltpu.VMEM((N,), jnp.float32),
    ],
)(_hist_body)


def _enc_body(x_ref, w_ref, b_ref, hist_ref, hs_ref, dis_ref):
    deg = jnp.sum(hist_ref[...], axis=0) + 1.0
    dis = deg ** -0.5
    h = lax.dot_general(x_ref[...], w_ref[...],
                        (((1,), (1,)), ((), ())),
                        precision=lax.Precision.HIGHEST,
                        preferred_element_type=jnp.float32)
    h = jnp.maximum(h + b_ref[...], 0.0)
    hs_ref[...] = h * dis[:, None]
    dis_ref[...] = dis[:, None]


def _enc(x, w, b, hist):
    return pl.pallas_call(
        _enc_body,
        out_shape=(jax.ShapeDtypeStruct((N, D), jnp.float32),
                   jax.ShapeDtypeStruct((N, 1), jnp.float32)),
    )(x, w, b, hist)


def _scat_body(hs_hbm, idx_hbm, zero_hbm, out_hbm,
               idx_v, rows_v, acc, gsems, ssems, isem):
    c = lax.axis_index("c")
    s = lax.axis_index("s")
    wid = c * NS + s
    base = s * CP
    pltpu.sync_copy(zero_hbm, rows_v.at[0])
    nz = CP // K
    rz = CP - nz * K

    def _zbody(i, carry):
        pltpu.sync_copy(rows_v.at[0], acc.at[pl.ds(base + i * K, K)])
        return carry
    lax.fori_loop(0, nz, _zbody, 0)
    pltpu.sync_copy(rows_v.at[0, pl.ds(0, rz)],
                    acc.at[pl.ds(base + nz * K, rz)])

    @pl.when(s == 0)
    def _():
        pltpu.sync_copy(rows_v.at[0, pl.ds(0, REM)],
                        acc.at[pl.ds(CP * NS, REM)])

    def _chunk_fire(ch, sync=False):
        r = (wid * NCHUNK + ch) * 2
        p = (ch % 2) * 2
        if sync:
            pltpu.sync_copy(idx_hbm.at[r], idx_v.at[p])
            pltpu.sync_copy(idx_hbm.at[r + 1], idx_v.at[p + 1])
        else:
            pltpu.async_copy(idx_hbm.at[r], idx_v.at[p], isem)
            pltpu.async_copy(idx_hbm.at[r + 1], idx_v.at[p + 1], isem)

    def _chunk_wait(ch):
        r = (wid * NCHUNK + ch) * 2
        p = (ch % 2) * 2
        pltpu.make_async_copy(idx_hbm.at[r], idx_v.at[p], isem).wait()
        pltpu.make_async_copy(idx_hbm.at[r + 1], idx_v.at[p + 1], isem).wait()

    _chunk_fire(0, sync=True)
    _chunk_fire(1)
    plsc.subcore_barrier()

    def _rowidx(j):
        return idx_v.at[((j // W) % 2) * 2, j % W]

    def _colidx(j):
        return idx_v.at[((j // W) % 2) * 2 + 1, j % W]

    def _gfire(j, b):
        pltpu.async_copy(hs_hbm.at[_rowidx(j)], rows_v.at[b], gsems.at[b])

    def _gwait(j, b):
        pltpu.make_async_copy(hs_hbm.at[_rowidx(j)], rows_v.at[b],
                              gsems.at[b]).wait()

    def _sfire(j, b):
        pltpu.async_copy(rows_v.at[b], acc.at[_colidx(j)], ssems.at[b],
                         add=True)

    def _swait(j, b):
        pltpu.make_async_copy(rows_v.at[b], acc.at[_colidx(j)],
                              ssems.at[b]).wait()

    def _visit(j, b):
        _gwait(j, b)
        _sfire(j, b)
        bp = (b + NBUF - 1) % NBUF

        @pl.when(j >= 1)
        def _():
            _swait(j - 1, bp)

        jf = j + NBUF - 1

        @pl.when(jnp.logical_and(jf % W == 0, jf < NBLK))
        def _():
            _chunk_wait(jf // W)

        @pl.when(jf < NBLK)
        def _():
            _gfire(jf, bp)

        @pl.when(jnp.logical_and(
            j % W == 1,
            jnp.logical_and(j // W >= 1, j // W <= NCHUNK - 2)))
        def _():
            _chunk_fire(j // W + 1)

    for b in range(NBUF - 1):
        _gfire(b, b)

    def group(g, carry):
        for b in range(NBUF):
            _visit(g * NBUF + b, b)
        return carry
    nfull = (NBLK // NBUF) * NBUF
    lax.fori_loop(0, NBLK // NBUF, group, 0)
    for jt in range(nfull, NBLK):
        _visit(jnp.int32(jt), jt % NBUF)
    _swait(jnp.int32(NBLK - 1), (NBLK - 1) % NBUF)

    plsc.subcore_barrier()
    pltpu.sync_copy(acc.at[pl.ds(base, CP)],
                    out_hbm.at[c, pl.ds(base, CP)])

    @pl.when(s == 0)
    def _():
        pltpu.sync_copy(acc.at[pl.ds(CP * NS, REM)],
                        out_hbm.at[c, pl.ds(CP * NS, REM)])


_scat_kernel = functools.partial(
    pl.kernel,
    out_type=jax.ShapeDtypeStruct((NC, N, D), jnp.float32),
    mesh=_mesh,
    compiler_params=_sc_params,
    scratch_types=[
        pltpu.VMEM((4, W, K), jnp.int32),
        pltpu.VMEM((NBUF, K, D), jnp.float32),
        pltpu.VMEM_SHARED((N, D), jnp.float32),
        pltpu.SemaphoreType.DMA((NBUF,)),
        pltpu.SemaphoreType.DMA((NBUF,)),
        pltpu.SemaphoreType.DMA,
    ],
)(_scat_body)


def _dec_body(p_ref, hs_ref, dis_ref, gb_ref, w_ref, b_ref, y_ref):
    m = (p_ref[0] + p_ref[1] + hs_ref[...]) * dis_ref[...] + gb_ref[...]
    y = lax.dot_general(m, w_ref[...], (((1,), (1,)), ((), ())),
                        precision=lax.Precision.HIGHEST,
                        preferred_element_type=jnp.float32)
    y_ref[...] = jax.nn.sigmoid(y + b_ref[...])


def _dec(p, hs, dis, gb, w, b):
    return pl.pallas_call(
        _dec_body,
        out_shape=jax.ShapeDtypeStruct((N, D), jnp.float32),
    )(p, hs, dis, gb, w, b)


def kernel(x, edge_index, W_enc, b_enc, W_dec, b_dec, gcn_bias):
    row = edge_index[0]
    col = edge_index[1]
    col_flat = col.reshape(NW, EPW)
    idx3 = jnp.concatenate(
        [row.reshape(NW, NCHUNK, 1, W, K), col.reshape(NW, NCHUNK, 1, W, K)],
        axis=2).reshape(NW * NCHUNK * 2, W, K)

    hist = _hist_kernel(col_flat)
    hs, dis = _enc(x, W_enc, b_enc.reshape(1, D), hist)
    partials = _scat_kernel(hs, idx3, jnp.zeros((K, D), jnp.float32))
    return _dec(partials, hs, dis, gcn_bias.reshape(1, D), W_dec,
                b_dec.reshape(1, D))


# speedup vs baseline: 1.0540x; 1.0039x over previous
import functools

import jax
import jax.numpy as jnp
from jax import lax
from jax.experimental import pallas as pl
from jax.experimental.pallas import tpu as pltpu
from jax.experimental.pallas import tpu_sc as plsc

N = 10000
E = 320000
D = 128
NC = 2
NS = 16
NW = NC * NS
EPW = E // NW
K = 80
NBLK = EPW // K
NBUF = 3
W = 25
WK = W * K
NCHUNK = NBLK // W
CP = 624
REM = N - CP * NS

_mesh = plsc.VectorSubcoreMesh(core_axis_name="c", subcore_axis_name="s")
_sc_params = pltpu.CompilerParams(needs_layout_passes=False)


def _hist_body(col_hbm, hist_out, col_v, hist_v):
    c = lax.axis_index("c")
    s = lax.axis_index("s")
    wid = c * NS + s
    pltpu.sync_copy(col_hbm.at[wid], col_v)

    def zero(i, carry):
        hist_v[pl.ds(pl.multiple_of(i * 16, 16), 16)] = jnp.zeros(
            (16,), jnp.float32)
        return carry
    lax.fori_loop(0, N // 16, zero, 0)

    ones = jnp.ones((16,), jnp.float32)

    def body(i, carry):
        cv = col_v[pl.ds(pl.multiple_of(i * 16, 16), 16)]
        plsc.addupdate_scatter(hist_v, [cv], ones)
        return carry
    lax.fori_loop(0, EPW // 16, body, 0)
    pltpu.sync_copy(hist_v, hist_out.at[wid])


_hist_kernel = functools.partial(
    pl.kernel,
    out_type=jax.ShapeDtypeStruct((NW, N), jnp.float32),
    mesh=_mesh,
    compiler_params=_sc_params,
    scratch_types=[
        pltpu.VMEM((EPW,), jnp.int32),
        pltpu.VMEM((N,), jnp.float32),
    ],
)(_hist_body)


def _enc_body(x_ref, w_ref, b_ref, hist_ref, hs_ref, dis_ref):
    deg = jnp.sum(hist_ref[...], axis=0) + 1.0
    dis = deg ** -0.5
    h = lax.dot_general(x_ref[...], w_ref[...],
                        (((1,), (1,)), ((), ())),
                        precision=lax.Precision.HIGHEST,
                        preferred_element_type=jnp.float32)
    h = jnp.maximum(h + b_ref[...], 0.0)
    hs_ref[...] = h * dis[:, None]
    dis_ref[...] = dis[:, None]


def _enc(x, w, b, hist):
    return pl.pallas_call(
        _enc_body,
        out_shape=(jax.ShapeDtypeStruct((N, D), jnp.float32),
                   jax.ShapeDtypeStruct((N, 1), jnp.float32)),
    )(x, w, b, hist)


def _scat_body(hs_hbm, idx_hbm, zero_hbm, out_hbm,
               idx_v, rows_v, acc, gsems, ssems, isem):
    c = lax.axis_index("c")
    s = lax.axis_index("s")
    wid = c * NS + s
    base = s * CP
    pltpu.sync_copy(zero_hbm, rows_v.at[NBUF - 1])

    def _chunk_fire(ch, sync=False):
        r = (wid * NCHUNK + ch) * 2
        p = (ch % 2) * 2
        if sync:
            pltpu.sync_copy(idx_hbm.at[r], idx_v.at[p])
            pltpu.sync_copy(idx_hbm.at[r + 1], idx_v.at[p + 1])
        else:
            pltpu.async_copy(idx_hbm.at[r], idx_v.at[p], isem)
            pltpu.async_copy(idx_hbm.at[r + 1], idx_v.at[p + 1], isem)

    def _chunk_wait(ch):
        r = (wid * NCHUNK + ch) * 2
        p = (ch % 2) * 2
        pltpu.make_async_copy(idx_hbm.at[r], idx_v.at[p], isem).wait()
        pltpu.make_async_copy(idx_hbm.at[r + 1], idx_v.at[p + 1], isem).wait()

    _chunk_fire(0, sync=True)
    _chunk_fire(1)

    def _rowidx(j):
        return idx_v.at[((j // W) % 2) * 2, j % W]

    def _colidx(j):
        return idx_v.at[((j // W) % 2) * 2 + 1, j % W]

    def _gfire(j, b):
        pltpu.async_copy(hs_hbm.at[_rowidx(j)], rows_v.at[b], gsems.at[b])

    def _gwait(j, b):
        pltpu.make_async_copy(hs_hbm.at[_rowidx(j)], rows_v.at[b],
                              gsems.at[b]).wait()

    def _sfire(j, b):
        pltpu.async_copy(rows_v.at[b], acc.at[_colidx(j)], ssems.at[b],
                         add=True)

    def _swait(j, b):
        pltpu.make_async_copy(rows_v.at[b], acc.at[_colidx(j)],
                              ssems.at[b]).wait()

    def _visit(j, b):
        _gwait(j, b)
        _sfire(j, b)
        bp = (b + NBUF - 1) % NBUF

        @pl.when(j >= 1)
        def _():
            _swait(j - 1, bp)

        jf = j + NBUF - 1

        @pl.when(jnp.logical_and(jf % W == 0, jf < NBLK))
        def _():
            _chunk_wait(jf // W)

        @pl.when(jf < NBLK)
        def _():
            _gfire(jf, bp)

        @pl.when(jnp.logical_and(
            j % W == 1,
            jnp.logical_and(j // W >= 1, j // W <= NCHUNK - 2)))
        def _():
            _chunk_fire(j // W + 1)

    for b in range(NBUF - 1):
        _gfire(b, b)
    nz = CP // K
    rz = CP - nz * K

    def _zbody(i, carry):
        pltpu.sync_copy(rows_v.at[NBUF - 1],
                        acc.at[pl.ds(base + i * K, K)])
        return carry
    lax.fori_loop(0, nz, _zbody, 0)
    pltpu.sync_copy(rows_v.at[NBUF - 1, pl.ds(0, rz)],
                    acc.at[pl.ds(base + nz * K, rz)])

    @pl.when(s == 0)
    def _():
        pltpu.sync_copy(rows_v.at[NBUF - 1, pl.ds(0, REM)],
                        acc.at[pl.ds(CP * NS, REM)])
    plsc.subcore_barrier()

    def group(g, carry):
        for b in range(NBUF):
            _visit(g * NBUF + b, b)
        return carry
    nfull = (NBLK // NBUF) * NBUF
    lax.fori_loop(0, NBLK // NBUF, group, 0)
    for jt in range(nfull, NBLK):
        _visit(jnp.int32(jt), jt % NBUF)
    _swait(jnp.int32(NBLK - 1), (NBLK - 1) % NBUF)

    plsc.subcore_barrier()
    pltpu.sync_copy(acc.at[pl.ds(base, CP)],
                    out_hbm.at[c, pl.ds(base, CP)])

    @pl.when(s == 0)
    def _():
        pltpu.sync_copy(acc.at[pl.ds(CP * NS, REM)],
                        out_hbm.at[c, pl.ds(CP * NS, REM)])


_scat_kernel = functools.partial(
    pl.kernel,
    out_type=jax.ShapeDtypeStruct((NC, N, D), jnp.float32),
    mesh=_mesh,
    compiler_params=_sc_params,
    scratch_types=[
        pltpu.VMEM((4, W, K), jnp.int32),
        pltpu.VMEM((NBUF, K, D), jnp.float32),
        pltpu.VMEM_SHARED((N, D), jnp.float32),
        pltpu.SemaphoreType.DMA((NBUF,)),
        pltpu.SemaphoreType.DMA((NBUF,)),
        pltpu.SemaphoreType.DMA,
    ],
)(_scat_body)


def _dec_body(p_ref, hs_ref, dis_ref, gb_ref, w_ref, b_ref, y_ref):
    m = (p_ref[0] + p_ref[1] + hs_ref[...]) * dis_ref[...] + gb_ref[...]
    y = lax.dot_general(m, w_ref[...], (((1,), (1,)), ((), ())),
                        precision=lax.Precision.HIGHEST,
                        preferred_element_type=jnp.float32)
    y_ref[...] = jax.nn.sigmoid(y + b_ref[...])


def _dec(p, hs, dis, gb, w, b):
    return pl.pallas_call(
        _dec_body,
        out_shape=jax.ShapeDtypeStruct((N, D), jnp.float32),
    )(p, hs, dis, gb, w, b)


def kernel(x, edge_index, W_enc, b_enc, W_dec, b_dec, gcn_bias):
    row = edge_index[0]
    col = edge_index[1]
    col_flat = col.reshape(NW, EPW)
    idx3 = jnp.concatenate(
        [row.reshape(NW, NCHUNK, 1, W, K), col.reshape(NW, NCHUNK, 1, W, K)],
        axis=2).reshape(NW * NCHUNK * 2, W, K)

    hist = _hist_kernel(col_flat)
    hs, dis = _enc(x, W_enc, b_enc.reshape(1, D), hist)
    partials = _scat_kernel(hs, idx3, jnp.zeros((K, D), jnp.float32))
    return _dec(partials, hs, dis, gcn_bias.reshape(1, D), W_dec,
                b_dec.reshape(1, D))
